# Initial kernel scaffold; baseline (speedup 1.0000x reference)
import jax
import jax.numpy as jnp
from jax.experimental import pallas as pl


def kernel(x, vec, edge_index, edge_rbf, weight, edge_vector, xp_w1, xp_b1, xp_w2, xp_b2, rbf_w, rbf_b, inv_w1, inv_b1, inv_w2, inv_b2):
    raise NotImplementedError("write your pallas kernel here")


# speedup vs baseline: 12.3422x; 12.3422x over previous
import functools
import math

import jax
import jax.numpy as jnp
from jax import lax
from jax.experimental import pallas as pl
from jax.experimental.pallas import tpu as pltpu
from jax.experimental.pallas import tpu_sc as plsc

N = 10000
E = 320000
H = 128
R = 32
INV_SQRT_3 = 1.0 / math.sqrt(3.0)
INV_SQRT_H = 1.0 / math.sqrt(H)

NW = 32
PER_W = E // NW
CG = 80
NCH_G = PER_W // CG

NS = 16
PER_S = E // NS
CS = 80
NCH_S = PER_S // CS
ZR = 400
NZCH = N // ZR

EB = 512


@functools.cache
def _sc_mesh():
    return plsc.VectorSubcoreMesh(
        core_axis_name="c", subcore_axis_name="s", num_cores=2, num_subcores=16
    )


@functools.cache
def _sc_gather_kernel():
    return pl.kernel(
        _sc_gather_body,
        mesh=_sc_mesh(),
        out_type=(
            jax.ShapeDtypeStruct((E, H), jnp.float32),
            jax.ShapeDtypeStruct((E, 3 * H), jnp.float32),
        ),
        scratch_types=[
            pltpu.VMEM((CG,), jnp.int32),
            pltpu.VMEM((CG, H), jnp.float32),
            pltpu.VMEM((CG, 3 * H), jnp.float32),
        ],
    )


def _sc_gather_body(x_hbm, vec_hbm, idx_hbm, xj_hbm, vj_hbm, idx_v, xrow_v, vrow_v):
    wid = lax.axis_index("s") * 2 + lax.axis_index("c")
    base = wid * PER_W

    @pl.loop(0, NCH_G)
    def _(i):
        b = base + i * CG
        pltpu.sync_copy(idx_hbm.at[pl.ds(b, CG)], idx_v)
        pltpu.sync_copy(x_hbm.at[idx_v], xrow_v)
        pltpu.sync_copy(vec_hbm.at[idx_v], vrow_v)
        pltpu.sync_copy(xrow_v, xj_hbm.at[pl.ds(b, CG)])
        pltpu.sync_copy(vrow_v, vj_hbm.at[pl.ds(b, CG)])


@functools.cache
def _sc_scatter_kernel():
    return pl.kernel(
        _sc_scatter_body,
        mesh=_sc_mesh(),
        out_type=jax.ShapeDtypeStruct((4, N, H), jnp.float32),
        scratch_types=[
            pltpu.VMEM((CS,), jnp.int32),
            pltpu.VMEM((CS, H), jnp.float32),
            pltpu.VMEM_SHARED((N, H), jnp.float32),
        ],
    )


def _sc_scatter_body(vals_hbm, idx_hbm, zero_hbm, out_hbm, idx_v, val_v, acc_sh):
    core = lax.axis_index("c")
    s = lax.axis_index("s")
    for p in range(2):
        plane = core * 2 + p
        for kk in range(2):
            k = s + NS * kk

            @pl.when(k < NZCH)
            def _():
                pltpu.sync_copy(zero_hbm, acc_sh.at[pl.ds(k * ZR, ZR)])

        plsc.subcore_barrier()

        base_e = s * PER_S

        @pl.loop(0, NCH_S)
        def _(i):
            b = base_e + i * CS
            pltpu.sync_copy(idx_hbm.at[pl.ds(b, CS)], idx_v)
            pltpu.sync_copy(vals_hbm.at[plane, pl.ds(b, CS)], val_v)
            pltpu.sync_copy(val_v, acc_sh.at[idx_v], add=True)

        plsc.subcore_barrier()

        for kk in range(2):
            k = s + NS * kk

            @pl.when(k < NZCH)
            def _():
                pltpu.sync_copy(
                    acc_sh.at[pl.ds(k * ZR, ZR)],
                    out_hbm.at[plane, pl.ds(k * ZR, ZR)],
                )


def _tc_dense_body(w_ref, rbf_ref, xj_ref, vj_ref, ev_ref,
                   xw1_ref, xb1_ref, xw2_ref, xb2_ref,
                   rw_ref, rb_ref, iw1_ref, ib1_ref, iw2_ref, ib2_ref,
                   out_ref):
    f32 = jnp.float32
    bf = jnp.bfloat16

    def mm(a, b):
        return jnp.dot(a.astype(bf), b.astype(bf), preferred_element_type=f32)

    h1 = mm(xj_ref[...], xw1_ref[...]) + xb1_ref[...]
    a1 = h1 * lax.logistic(h1)
    xh = mm(a1, xw2_ref[...]) + xb2_ref[...]
    rbfh = mm(rbf_ref[...], rw_ref[...]) + rb_ref[...]
    g1 = mm(w_ref[...], iw1_ref[...]) + ib1_ref[...]
    ga = g1 * lax.logistic(g1)
    g2 = mm(ga, iw2_ref[...]) + ib2_ref[...]
    m = xh * (rbfh * g2)
    x1 = m[:, :H]
    xh2 = m[:, H:2 * H] * INV_SQRT_3
    xh3 = m[:, 2 * H:]
    out_ref[0, :, :] = x1
    for c in range(3):
        ev_c = ev_ref[:, c][:, None]
        out_ref[1 + c, :, :] = (
            vj_ref[:, c * H:(c + 1) * H] * xh2 + ev_c * xh3
        ) * INV_SQRT_H


def _tc_dense(weight, edge_rbf, x_j, vec_j, ev,
              xp_w1, xp_b1, xp_w2, xp_b2, rbf_w, rbf_b,
              inv_w1, inv_b1, inv_w2, inv_b2):
    grid = (E // EB,)
    edge_spec = lambda d: pl.BlockSpec((EB, d), lambda i: (i, 0))
    full_spec = lambda a, b: pl.BlockSpec((a, b), lambda i: (0, 0))
    return pl.pallas_call(
        _tc_dense_body,
        grid=grid,
        in_specs=[
            edge_spec(3 * H + R),
            edge_spec(R),
            edge_spec(H),
            edge_spec(3 * H),
            edge_spec(3),
            full_spec(H, H),
            full_spec(1, H),
            full_spec(H, 3 * H),
            full_spec(1, 3 * H),
            full_spec(R, 3 * H),
            full_spec(1, 3 * H),
            full_spec(3 * H + R, 3 * H),
            full_spec(1, 3 * H),
            full_spec(3 * H, 3 * H),
            full_spec(1, 3 * H),
        ],
        out_specs=pl.BlockSpec((4, EB, H), lambda i: (0, i, 0)),
        out_shape=jax.ShapeDtypeStruct((4, E, H), jnp.float32),
    )(weight, edge_rbf, x_j, vec_j, ev,
      xp_w1, xp_b1.reshape(1, H), xp_w2, xp_b2.reshape(1, 3 * H),
      rbf_w, rbf_b.reshape(1, 3 * H),
      inv_w1, inv_b1.reshape(1, 3 * H), inv_w2, inv_b2.reshape(1, 3 * H))


@jax.jit
def _impl(x, vec, edge_index, edge_rbf, weight, edge_vector,
          xp_w1, xp_b1, xp_w2, xp_b2, rbf_w, rbf_b,
          inv_w1, inv_b1, inv_w2, inv_b2):
    src = edge_index[0].astype(jnp.int32)
    dst = edge_index[1].astype(jnp.int32)
    vec_flat = vec.reshape(N, 3 * H)
    x_j, vec_j = _sc_gather_kernel()(x, vec_flat, src)
    planes = _tc_dense(weight, edge_rbf, x_j, vec_j, edge_vector,
                       xp_w1, xp_b1, xp_w2, xp_b2, rbf_w, rbf_b,
                       inv_w1, inv_b1, inv_w2, inv_b2)
    zero = jnp.zeros((ZR, H), jnp.float32)
    acc = _sc_scatter_kernel()(planes, dst, zero)
    dx = acc[0]
    dvec = jnp.stack((acc[1], acc[2], acc[3]), axis=1)
    return dx, dvec


def kernel(x, vec, edge_index, edge_rbf, weight, edge_vector,
           xp_w1, xp_b1, xp_w2, xp_b2, rbf_w, rbf_b,
           inv_w1, inv_b1, inv_w2, inv_b2):
    return _impl(x, vec, edge_index, edge_rbf, weight, edge_vector,
                 xp_w1, xp_b1, xp_w2, xp_b2, rbf_w, rbf_b,
                 inv_w1, inv_b1, inv_w2, inv_b2)


# speedup vs baseline: 14.3636x; 1.1638x over previous
import functools
import math

import jax
import jax.numpy as jnp
from jax import lax
from jax.experimental import pallas as pl
from jax.experimental.pallas import tpu as pltpu
from jax.experimental.pallas import tpu_sc as plsc

N = 10000
E = 320000
H = 128
R = 32
INV_SQRT_3 = 1.0 / math.sqrt(3.0)
INV_SQRT_H = 1.0 / math.sqrt(H)

NW = 32
PER_W = E // NW
CG = 400
NCH_G = PER_W // CG

NS = 16
PER_S = E // NS
CS = 200
NCH_S = PER_S // CS
ZR = 400
NZCH = N // ZR

EB = 512


@functools.cache
def _sc_mesh():
    return plsc.VectorSubcoreMesh(
        core_axis_name="c", subcore_axis_name="s", num_cores=2, num_subcores=16
    )


@functools.cache
def _sc_gather_kernel():
    return pl.kernel(
        _sc_gather_body,
        mesh=_sc_mesh(),
        out_type=jax.ShapeDtypeStruct((E, 2 * H), jnp.float32),
        scratch_types=[
            pltpu.VMEM((CG,), jnp.int32),
            pltpu.VMEM((CG, 2 * H), jnp.float32),
        ],
    )


def _sc_gather_body(tab_hbm, idx_hbm, out_hbm, idx_v, row_v):
    wid = lax.axis_index("s") * 2 + lax.axis_index("c")
    base = wid * PER_W

    @pl.loop(0, NCH_G)
    def _(i):
        b = base + i * CG
        pltpu.sync_copy(idx_hbm.at[pl.ds(b, CG)], idx_v)
        pltpu.sync_copy(tab_hbm.at[idx_v], row_v)
        pltpu.sync_copy(row_v, out_hbm.at[pl.ds(b, CG)])


@functools.cache
def _sc_scatter_kernel():
    return pl.kernel(
        _sc_scatter_body,
        mesh=_sc_mesh(),
        out_type=jax.ShapeDtypeStruct((4, N, H), jnp.float32),
        scratch_types=[
            pltpu.VMEM((CS,), jnp.int32),
            pltpu.VMEM((CS, H), jnp.float32),
            pltpu.VMEM_SHARED((N, H), jnp.float32),
        ],
    )


def _sc_scatter_body(vals_hbm, idx_hbm, zero_hbm, out_hbm, idx_v, val_v, acc_sh):
    core = lax.axis_index("c")
    s = lax.axis_index("s")
    for p in range(2):
        plane = core * 2 + p
        for kk in range(2):
            k = s + NS * kk

            @pl.when(k < NZCH)
            def _():
                pltpu.sync_copy(zero_hbm, acc_sh.at[pl.ds(k * ZR, ZR)])

        plsc.subcore_barrier()

        base_e = s * PER_S

        @pl.loop(0, NCH_S)
        def _(i):
            b = base_e + i * CS
            pltpu.sync_copy(idx_hbm.at[pl.ds(b, CS)], idx_v)
            pltpu.sync_copy(vals_hbm.at[plane, pl.ds(b, CS)], val_v)
            pltpu.sync_copy(val_v, acc_sh.at[idx_v], add=True)

        plsc.subcore_barrier()

        for kk in range(2):
            k = s + NS * kk

            @pl.when(k < NZCH)
            def _():
                pltpu.sync_copy(
                    acc_sh.at[pl.ds(k * ZR, ZR)],
                    out_hbm.at[plane, pl.ds(k * ZR, ZR)],
                )


def _pack_pairs(a_bf):
    k = a_bf.shape[1] // 2
    lo = lax.bitcast_convert_type(a_bf[:, :k], jnp.uint16).astype(jnp.uint32)
    hi = lax.bitcast_convert_type(a_bf[:, k:], jnp.uint16).astype(jnp.uint32)
    return lax.bitcast_convert_type((hi << 16) | lo, jnp.float32)


def _unpack_pairs(p):
    xi = lax.bitcast_convert_type(p, jnp.int32)
    lo = lax.bitcast_convert_type(xi << 16, jnp.float32)
    hi = lax.bitcast_convert_type(xi & jnp.int32(-65536), jnp.float32)
    return jnp.concatenate([lo, hi], axis=1)


def _tc_dense_body(w_ref, rbf_ref, xvj_ref, ev_ref,
                   xw1_ref, xb1_ref, xw2_ref, xb2_ref,
                   rw_ref, rb_ref, iw1_ref, ib1_ref, iw2_ref, ib2_ref,
                   out_ref):
    f32 = jnp.float32
    bf = jnp.bfloat16

    def mm(a, b):
        return jnp.dot(a.astype(bf), b.astype(bf), preferred_element_type=f32)

    xj = _unpack_pairs(xvj_ref[:, :H // 2])
    vj = _unpack_pairs(xvj_ref[:, H // 2:])
    h1 = mm(xj, xw1_ref[...]) + xb1_ref[...]
    a1 = h1 * lax.logistic(h1)
    xh = mm(a1, xw2_ref[...]) + xb2_ref[...]
    rbfh = mm(rbf_ref[...], rw_ref[...]) + rb_ref[...]
    g1 = mm(w_ref[...], iw1_ref[...]) + ib1_ref[...]
    ga = g1 * lax.logistic(g1)
    g2 = mm(ga, iw2_ref[...]) + ib2_ref[...]
    m = xh * (rbfh * g2)
    x1 = m[:, :H]
    xh2 = m[:, H:2 * H] * INV_SQRT_3
    xh3 = m[:, 2 * H:]
    out_ref[0, :, :] = x1
    for c in range(3):
        ev_c = ev_ref[:, c][:, None]
        vj_c = vj[:, c * H:(c + 1) * H]
        out_ref[1 + c, :, :] = (vj_c * xh2 + ev_c * xh3) * INV_SQRT_H


def _tc_dense(weight, edge_rbf, xv_j, ev,
              xp_w1, xp_b1, xp_w2, xp_b2, rbf_w, rbf_b,
              inv_w1, inv_b1, inv_w2, inv_b2):
    grid = (E // EB,)
    edge_spec = lambda d: pl.BlockSpec((EB, d), lambda i: (i, 0))
    full_spec = lambda a, b: pl.BlockSpec((a, b), lambda i: (0, 0))
    return pl.pallas_call(
        _tc_dense_body,
        grid=grid,
        in_specs=[
            edge_spec(3 * H + R),
            edge_spec(R),
            edge_spec(2 * H),
            edge_spec(3),
            full_spec(H, H),
            full_spec(1, H),
            full_spec(H, 3 * H),
            full_spec(1, 3 * H),
            full_spec(R, 3 * H),
            full_spec(1, 3 * H),
            full_spec(3 * H + R, 3 * H),
            full_spec(1, 3 * H),
            full_spec(3 * H, 3 * H),
            full_spec(1, 3 * H),
        ],
        out_specs=pl.BlockSpec((4, EB, H), lambda i: (0, i, 0)),
        out_shape=jax.ShapeDtypeStruct((4, E, H), jnp.float32),
        compiler_params=pltpu.CompilerParams(
            dimension_semantics=("parallel",)),
    )(weight, edge_rbf, xv_j, ev,
      xp_w1, xp_b1.reshape(1, H), xp_w2, xp_b2.reshape(1, 3 * H),
      rbf_w, rbf_b.reshape(1, 3 * H),
      inv_w1, inv_b1.reshape(1, 3 * H), inv_w2, inv_b2.reshape(1, 3 * H))


@jax.jit
def _impl(x, vec, edge_index, edge_rbf, weight, edge_vector,
          xp_w1, xp_b1, xp_w2, xp_b2, rbf_w, rbf_b,
          inv_w1, inv_b1, inv_w2, inv_b2):
    src = edge_index[0].astype(jnp.int32)
    dst = edge_index[1].astype(jnp.int32)
    vec_flat = vec.reshape(N, 3 * H)
    x_pk = _pack_pairs(x.astype(jnp.bfloat16))
    vec_pk = _pack_pairs(vec_flat.astype(jnp.bfloat16))
    tab = jnp.concatenate([x_pk, vec_pk], axis=1)
    xv_j = _sc_gather_kernel()(tab, src)
    planes = _tc_dense(weight, edge_rbf, xv_j, edge_vector,
                       xp_w1, xp_b1, xp_w2, xp_b2, rbf_w, rbf_b,
                       inv_w1, inv_b1, inv_w2, inv_b2)
    zero = jnp.zeros((ZR, H), jnp.float32)
    acc = _sc_scatter_kernel()(planes, dst, zero)
    dx = acc[0]
    dvec = jnp.stack((acc[1], acc[2], acc[3]), axis=1)
    return dx, dvec


def kernel(x, vec, edge_index, edge_rbf, weight, edge_vector,
           xp_w1, xp_b1, xp_w2, xp_b2, rbf_w, rbf_b,
           inv_w1, inv_b1, inv_w2, inv_b2):
    return _impl(x, vec, edge_index, edge_rbf, weight, edge_vector,
                 xp_w1, xp_b1, xp_w2, xp_b2, rbf_w, rbf_b,
                 inv_w1, inv_b1, inv_w2, inv_b2)


# speedup vs baseline: 18.9862x; 1.3218x over previous
import functools
import math

import jax
import jax.numpy as jnp
from jax import lax
from jax.experimental import pallas as pl
from jax.experimental.pallas import tpu as pltpu
from jax.experimental.pallas import tpu_sc as plsc

N = 10000
E = 320000
H = 128
R = 32
INV_SQRT_3 = 1.0 / math.sqrt(3.0)
INV_SQRT_H = 1.0 / math.sqrt(H)

NW = 32
PER_W = E // NW
CG = 400
NCH_G = PER_W // CG

NS = 16
PER_S = E // NS
CS = 200
NCH_S = PER_S // CS
ZR = 400
NZCH = N // ZR

EB = 512


@functools.cache
def _sc_mesh():
    return plsc.VectorSubcoreMesh(
        core_axis_name="c", subcore_axis_name="s", num_cores=2, num_subcores=16
    )


@functools.cache
def _sc_gather_kernel():
    return pl.kernel(
        _sc_gather_body,
        mesh=_sc_mesh(),
        out_type=jax.ShapeDtypeStruct((E, 2 * H), jnp.float32),
        scratch_types=[
            pltpu.VMEM((CG,), jnp.int32),
            pltpu.VMEM((CG, 2 * H), jnp.float32),
        ],
    )


def _sc_gather_body(tab_hbm, idx_hbm, out_hbm, idx_v, row_v):
    wid = lax.axis_index("s") * 2 + lax.axis_index("c")
    base = wid * PER_W

    @pl.loop(0, NCH_G)
    def _(i):
        b = base + i * CG
        pltpu.sync_copy(idx_hbm.at[pl.ds(b, CG)], idx_v)
        pltpu.sync_copy(tab_hbm.at[idx_v], row_v)
        pltpu.sync_copy(row_v, out_hbm.at[pl.ds(b, CG)])


@functools.cache
def _sc_scatter_kernel():
    return pl.kernel(
        _sc_scatter_body,
        mesh=_sc_mesh(),
        out_type=jax.ShapeDtypeStruct((4, N, H), jnp.float32),
        scratch_types=[
            pltpu.VMEM((CS,), jnp.int32),
            pltpu.VMEM((CS, H), jnp.float32),
            pltpu.VMEM_SHARED((N, H), jnp.float32),
        ],
    )


def _sc_scatter_body(vals_hbm, idx_hbm, zero_hbm, out_hbm, idx_v, val_v, acc_sh):
    core = lax.axis_index("c")
    s = lax.axis_index("s")
    for p in range(2):
        plane = core * 2 + p
        for kk in range(2):
            k = s + NS * kk

            @pl.when(k < NZCH)
            def _():
                pltpu.sync_copy(zero_hbm, acc_sh.at[pl.ds(k * ZR, ZR)])

        plsc.subcore_barrier()

        base_e = s * PER_S

        @pl.loop(0, NCH_S)
        def _(i):
            b = base_e + i * CS
            pltpu.sync_copy(idx_hbm.at[pl.ds(b, CS)], idx_v)
            pltpu.sync_copy(vals_hbm.at[plane, pl.ds(b, CS)], val_v)
            pltpu.sync_copy(val_v, acc_sh.at[idx_v], add=True)

        plsc.subcore_barrier()

        for kk in range(2):
            k = s + NS * kk

            @pl.when(k < NZCH)
            def _():
                pltpu.sync_copy(
                    acc_sh.at[pl.ds(k * ZR, ZR)],
                    out_hbm.at[plane, pl.ds(k * ZR, ZR)],
                )


def _pack_pairs(a_bf):
    k = a_bf.shape[1] // 2
    lo = lax.bitcast_convert_type(a_bf[:, :k], jnp.uint16).astype(jnp.uint32)
    hi = lax.bitcast_convert_type(a_bf[:, k:], jnp.uint16).astype(jnp.uint32)
    return lax.bitcast_convert_type((hi << 16) | lo, jnp.float32)


def _unpack_pairs(p):
    xi = lax.bitcast_convert_type(p, jnp.int32)
    lo = lax.bitcast_convert_type(xi << 16, jnp.float32)
    hi = lax.bitcast_convert_type(xi & jnp.int32(-65536), jnp.float32)
    return jnp.concatenate([lo, hi], axis=1)


def _tc_dense_body(wT_ref, rbfT_ref, xvj_ref, evT_ref,
                   xw1_ref, xb1_ref, xw2_ref, xb2_ref,
                   rw_ref, rb_ref, iw1_ref, ib1_ref, iw2_ref, ib2_ref,
                   out_ref):
    f32 = jnp.float32
    bf = jnp.bfloat16

    def mm(a, b):
        return jnp.dot(a.astype(bf), b.astype(bf), preferred_element_type=f32)

    def mm_t(a, b):
        return lax.dot_general(a.astype(bf), b.astype(bf),
                               (((0,), (0,)), ((), ())),
                               preferred_element_type=f32)

    xj = _unpack_pairs(xvj_ref[:, :H // 2])
    vj = _unpack_pairs(xvj_ref[:, H // 2:])
    h1 = mm(xj, xw1_ref[...]) + xb1_ref[...]
    a1 = h1 * lax.logistic(h1)
    xh = mm(a1, xw2_ref[...]) + xb2_ref[...]
    rbfh = mm_t(rbfT_ref[...], rw_ref[...]) + rb_ref[...]
    g1 = mm_t(wT_ref[...], iw1_ref[...]) + ib1_ref[...]
    ga = g1 * lax.logistic(g1)
    g2 = mm(ga, iw2_ref[...]) + ib2_ref[...]
    m = xh * (rbfh * g2)
    x1 = m[:, :H]
    xh2 = m[:, H:2 * H] * INV_SQRT_3
    xh3 = m[:, 2 * H:]
    out_ref[0, :, :] = x1
    for c in range(3):
        ev_c = jnp.transpose(evT_ref[c:c + 1, :])
        vj_c = vj[:, c * H:(c + 1) * H]
        out_ref[1 + c, :, :] = (vj_c * xh2 + ev_c * xh3) * INV_SQRT_H


def _tc_dense(weightT, edge_rbfT, xv_j, evT,
              xp_w1, xp_b1, xp_w2, xp_b2, rbf_w, rbf_b,
              inv_w1, inv_b1, inv_w2, inv_b2):
    grid = (E // EB,)
    edge_spec = lambda d: pl.BlockSpec((EB, d), lambda i: (i, 0))
    edge_spec_t = lambda d: pl.BlockSpec((d, EB), lambda i: (0, i))
    full_spec = lambda a, b: pl.BlockSpec((a, b), lambda i: (0, 0))
    return pl.pallas_call(
        _tc_dense_body,
        grid=grid,
        in_specs=[
            edge_spec_t(3 * H + R),
            edge_spec_t(R),
            edge_spec(2 * H),
            edge_spec_t(3),
            full_spec(H, H),
            full_spec(1, H),
            full_spec(H, 3 * H),
            full_spec(1, 3 * H),
            full_spec(R, 3 * H),
            full_spec(1, 3 * H),
            full_spec(3 * H + R, 3 * H),
            full_spec(1, 3 * H),
            full_spec(3 * H, 3 * H),
            full_spec(1, 3 * H),
        ],
        out_specs=pl.BlockSpec((4, EB, H), lambda i: (0, i, 0)),
        out_shape=jax.ShapeDtypeStruct((4, E, H), jnp.float32),
        compiler_params=pltpu.CompilerParams(
            dimension_semantics=("parallel",)),
    )(weightT, edge_rbfT, xv_j, evT,
      xp_w1, xp_b1.reshape(1, H), xp_w2, xp_b2.reshape(1, 3 * H),
      rbf_w, rbf_b.reshape(1, 3 * H),
      inv_w1, inv_b1.reshape(1, 3 * H), inv_w2, inv_b2.reshape(1, 3 * H))


@jax.jit
def _impl(x, vec, edge_index, edge_rbf, weight, edge_vector,
          xp_w1, xp_b1, xp_w2, xp_b2, rbf_w, rbf_b,
          inv_w1, inv_b1, inv_w2, inv_b2):
    src = edge_index[0].astype(jnp.int32)
    dst = edge_index[1].astype(jnp.int32)
    vec_flat = vec.reshape(N, 3 * H)
    x_pk = _pack_pairs(x.astype(jnp.bfloat16))
    vec_pk = _pack_pairs(vec_flat.astype(jnp.bfloat16))
    tab = jnp.concatenate([x_pk, vec_pk], axis=1)
    xv_j = _sc_gather_kernel()(tab, src)
    planes = _tc_dense(weight.T, edge_rbf.T, xv_j, edge_vector.T,
                       xp_w1, xp_b1, xp_w2, xp_b2, rbf_w, rbf_b,
                       inv_w1, inv_b1, inv_w2, inv_b2)
    zero = jnp.zeros((ZR, H), jnp.float32)
    acc = _sc_scatter_kernel()(planes, dst, zero)
    dx = acc[0]
    dvec = jnp.stack((acc[1], acc[2], acc[3]), axis=1)
    return dx, dvec


def kernel(x, vec, edge_index, edge_rbf, weight, edge_vector,
           xp_w1, xp_b1, xp_w2, xp_b2, rbf_w, rbf_b,
           inv_w1, inv_b1, inv_w2, inv_b2):
    return _impl(x, vec, edge_index, edge_rbf, weight, edge_vector,
                 xp_w1, xp_b1, xp_w2, xp_b2, rbf_w, rbf_b,
                 inv_w1, inv_b1, inv_w2, inv_b2)


# speedup vs baseline: 26.0138x; 1.3701x over previous
import functools
import math

import jax
import jax.numpy as jnp
from jax import lax
from jax.experimental import pallas as pl
from jax.experimental.pallas import tpu as pltpu
from jax.experimental.pallas import tpu_sc as plsc

N = 10000
E = 320000
H = 128
R = 32
INV_SQRT_3 = 1.0 / math.sqrt(3.0)
INV_SQRT_H = 1.0 / math.sqrt(H)

NW = 32
CG = 400

NS = 16
CS = 200
ZR = 400
NZCH = N // ZR

EB = 512
NSEG = 5
SEG = E // NSEG


@functools.cache
def _sc_mesh():
    return plsc.VectorSubcoreMesh(
        core_axis_name="c", subcore_axis_name="s", num_cores=2, num_subcores=16
    )


@functools.cache
def _sc_gather_kernel(seg):
    per_w = seg // NW
    nch = per_w // CG

    def body(tab_hbm, idx_hbm, out_hbm, idx_v, row_v):
        wid = lax.axis_index("s") * 2 + lax.axis_index("c")
        base = wid * per_w

        @pl.loop(0, nch)
        def _(i):
            b = base + i * CG
            pltpu.sync_copy(idx_hbm.at[pl.ds(b, CG)], idx_v)
            pltpu.sync_copy(tab_hbm.at[idx_v], row_v)
            pltpu.sync_copy(row_v, out_hbm.at[pl.ds(b, CG)])

    return pl.kernel(
        body,
        mesh=_sc_mesh(),
        out_type=jax.ShapeDtypeStruct((seg, 2 * H), jnp.float32),
        scratch_types=[
            pltpu.VMEM((CG,), jnp.int32),
            pltpu.VMEM((CG, 2 * H), jnp.float32),
        ],
    )


@functools.cache
def _sc_scatter_kernel(seg):
    per_s = seg // NS
    nch = per_s // CS

    def body(vals_hbm, idx_hbm, init_hbm, out_hbm, idx_v, val_v, acc_sh):
        core = lax.axis_index("c")
        s = lax.axis_index("s")
        for p in range(2):
            plane = core * 2 + p
            for kk in range(2):
                k = s + NS * kk

                @pl.when(k < NZCH)
                def _():
                    pltpu.sync_copy(init_hbm.at[plane, pl.ds(k * ZR, ZR)],
                                    acc_sh.at[pl.ds(k * ZR, ZR)])

            plsc.subcore_barrier()

            base_e = s * per_s

            @pl.loop(0, nch)
            def _(i):
                b = base_e + i * CS
                pltpu.sync_copy(idx_hbm.at[pl.ds(b, CS)], idx_v)
                pltpu.sync_copy(vals_hbm.at[plane, pl.ds(b, CS)], val_v)
                pltpu.sync_copy(val_v, acc_sh.at[idx_v], add=True)

            plsc.subcore_barrier()

            for kk in range(2):
                k = s + NS * kk

                @pl.when(k < NZCH)
                def _():
                    pltpu.sync_copy(
                        acc_sh.at[pl.ds(k * ZR, ZR)],
                        out_hbm.at[plane, pl.ds(k * ZR, ZR)],
                    )

    return pl.kernel(
        body,
        mesh=_sc_mesh(),
        out_type=jax.ShapeDtypeStruct((4, N, H), jnp.float32),
        scratch_types=[
            pltpu.VMEM((CS,), jnp.int32),
            pltpu.VMEM((CS, H), jnp.float32),
            pltpu.VMEM_SHARED((N, H), jnp.float32),
        ],
    )


def _pack_pairs(a_bf):
    k = a_bf.shape[1] // 2
    lo = lax.bitcast_convert_type(a_bf[:, :k], jnp.uint16).astype(jnp.uint32)
    hi = lax.bitcast_convert_type(a_bf[:, k:], jnp.uint16).astype(jnp.uint32)
    return lax.bitcast_convert_type((hi << 16) | lo, jnp.float32)


def _unpack_pairs(p):
    xi = lax.bitcast_convert_type(p, jnp.int32)
    lo = lax.bitcast_convert_type(xi << 16, jnp.float32)
    hi = lax.bitcast_convert_type(xi & jnp.int32(-65536), jnp.float32)
    return jnp.concatenate([lo, hi], axis=1)


def _tc_dense_body(wT_ref, rbfT_ref, xvj_ref, evT_ref,
                   xw1_ref, xb1_ref, xw2_ref, xb2_ref,
                   rw_ref, rb_ref, iw1_ref, ib1_ref, iw2_ref, ib2_ref,
                   out_ref):
    f32 = jnp.float32
    bf = jnp.bfloat16

    def mm(a, b):
        return jnp.dot(a.astype(bf), b.astype(bf), preferred_element_type=f32)

    def mm_t(a, b):
        return lax.dot_general(a.astype(bf), b.astype(bf),
                               (((0,), (0,)), ((), ())),
                               preferred_element_type=f32)

    xj = _unpack_pairs(xvj_ref[:, :H // 2])
    vj = _unpack_pairs(xvj_ref[:, H // 2:])
    h1 = mm(xj, xw1_ref[...]) + xb1_ref[...]
    a1 = h1 * lax.logistic(h1)
    xh = mm(a1, xw2_ref[...]) + xb2_ref[...]
    rbfh = mm_t(rbfT_ref[...], rw_ref[...]) + rb_ref[...]
    g1 = mm_t(wT_ref[...], iw1_ref[...]) + ib1_ref[...]
    ga = g1 * lax.logistic(g1)
    g2 = mm(ga, iw2_ref[...]) + ib2_ref[...]
    m = xh * (rbfh * g2)
    x1 = m[:, :H]
    xh2 = m[:, H:2 * H] * INV_SQRT_3
    xh3 = m[:, 2 * H:]
    out_ref[0, :, :] = x1
    for c in range(3):
        ev_c = jnp.transpose(evT_ref[c:c + 1, :])
        vj_c = vj[:, c * H:(c + 1) * H]
        out_ref[1 + c, :, :] = (vj_c * xh2 + ev_c * xh3) * INV_SQRT_H


def _tc_dense(base, seg, weightT, edge_rbfT, xv_j, evT,
              xp_w1, xp_b1, xp_w2, xp_b2, rbf_w, rbf_b,
              inv_w1, inv_b1, inv_w2, inv_b2):
    grid = (seg // EB,)
    bb = base // EB
    edge_spec = lambda d: pl.BlockSpec((EB, d), lambda i: (i, 0))
    edge_spec_t = lambda d: pl.BlockSpec((d, EB), lambda i: (0, i + bb))
    full_spec = lambda a, b: pl.BlockSpec((a, b), lambda i: (0, 0))
    return pl.pallas_call(
        _tc_dense_body,
        grid=grid,
        in_specs=[
            edge_spec_t(3 * H + R),
            edge_spec_t(R),
            edge_spec(2 * H),
            edge_spec_t(3),
            full_spec(H, H),
            full_spec(1, H),
            full_spec(H, 3 * H),
            full_spec(1, 3 * H),
            full_spec(R, 3 * H),
            full_spec(1, 3 * H),
            full_spec(3 * H + R, 3 * H),
            full_spec(1, 3 * H),
            full_spec(3 * H, 3 * H),
            full_spec(1, 3 * H),
        ],
        out_specs=pl.BlockSpec((4, EB, H), lambda i: (0, i, 0)),
        out_shape=jax.ShapeDtypeStruct((4, seg, H), jnp.float32),
        compiler_params=pltpu.CompilerParams(
            dimension_semantics=("parallel",)),
    )(weightT, edge_rbfT, xv_j, evT,
      xp_w1, xp_b1.reshape(1, H), xp_w2, xp_b2.reshape(1, 3 * H),
      rbf_w, rbf_b.reshape(1, 3 * H),
      inv_w1, inv_b1.reshape(1, 3 * H), inv_w2, inv_b2.reshape(1, 3 * H))


@jax.jit
def _impl(x, vec, edge_index, edge_rbf, weight, edge_vector,
          xp_w1, xp_b1, xp_w2, xp_b2, rbf_w, rbf_b,
          inv_w1, inv_b1, inv_w2, inv_b2):
    src = edge_index[0].astype(jnp.int32)
    dst = edge_index[1].astype(jnp.int32)
    vec_flat = vec.reshape(N, 3 * H)
    x_pk = _pack_pairs(x.astype(jnp.bfloat16))
    vec_pk = _pack_pairs(vec_flat.astype(jnp.bfloat16))
    tab = jnp.concatenate([x_pk, vec_pk], axis=1)
    acc = jnp.zeros((4, N, H), jnp.float32)
    for k in range(NSEG):
        sl = slice(k * SEG, (k + 1) * SEG)
        xv_k = _sc_gather_kernel(SEG)(tab, src[sl])
        planes_k = _tc_dense(k * SEG, SEG, weight.T, edge_rbf.T, xv_k,
                             edge_vector.T,
                             xp_w1, xp_b1, xp_w2, xp_b2, rbf_w, rbf_b,
                             inv_w1, inv_b1, inv_w2, inv_b2)
        acc = _sc_scatter_kernel(SEG)(planes_k, dst[sl], acc)
    dx = acc[0]
    dvec = jnp.stack((acc[1], acc[2], acc[3]), axis=1)
    return dx, dvec


def kernel(x, vec, edge_index, edge_rbf, weight, edge_vector,
           xp_w1, xp_b1, xp_w2, xp_b2, rbf_w, rbf_b,
           inv_w1, inv_b1, inv_w2, inv_b2):
    return _impl(x, vec, edge_index, edge_rbf, weight, edge_vector,
                 xp_w1, xp_b1, xp_w2, xp_b2, rbf_w, rbf_b,
                 inv_w1, inv_b1, inv_w2, inv_b2)


# speedup vs baseline: 26.8438x; 1.0319x over previous
import functools
import math

import jax
import jax.numpy as jnp
from jax import lax
from jax.experimental import pallas as pl
from jax.experimental.pallas import tpu as pltpu
from jax.experimental.pallas import tpu_sc as plsc

N = 10000
E = 320000
H = 128
R = 32
INV_SQRT_3 = 1.0 / math.sqrt(3.0)
INV_SQRT_H = 1.0 / math.sqrt(H)

NW = 32
CG = 400

NS = 16
CS = 200
ZR = 400
NZCH = N // ZR

EB = 640
SEGS = (12800, 64000, 76800, 76800, 76800, 12800)


@functools.cache
def _sc_mesh():
    return plsc.VectorSubcoreMesh(
        core_axis_name="c", subcore_axis_name="s", num_cores=2, num_subcores=16
    )


@functools.cache
def _sc_gather_kernel(seg):
    per_w = seg // NW
    nch = per_w // CG

    def body(tab_hbm, idx_hbm, out_hbm, idx_v, row_v):
        wid = lax.axis_index("s") * 2 + lax.axis_index("c")
        base = wid * per_w

        @pl.loop(0, nch)
        def _(i):
            b = base + i * CG
            pltpu.sync_copy(idx_hbm.at[pl.ds(b, CG)], idx_v)
            pltpu.sync_copy(tab_hbm.at[idx_v], row_v)
            pltpu.sync_copy(row_v, out_hbm.at[pl.ds(b, CG)])

    return pl.kernel(
        body,
        mesh=_sc_mesh(),
        out_type=jax.ShapeDtypeStruct((seg, 2 * H), jnp.float32),
        scratch_types=[
            pltpu.VMEM((CG,), jnp.int32),
            pltpu.VMEM((CG, 2 * H), jnp.float32),
        ],
    )


@functools.cache
def _sc_scatter_kernel(seg):
    per_s = seg // NS
    nch = per_s // CS

    def body(vals_hbm, idx_hbm, init_hbm, out_hbm, idx_v, val_v, acc_sh):
        core = lax.axis_index("c")
        s = lax.axis_index("s")
        for p in range(2):
            plane = core * 2 + p
            for kk in range(2):
                k = s + NS * kk

                @pl.when(k < NZCH)
                def _():
                    pltpu.sync_copy(init_hbm.at[plane, pl.ds(k * ZR, ZR)],
                                    acc_sh.at[pl.ds(k * ZR, ZR)])

            plsc.subcore_barrier()

            base_e = s * per_s

            @pl.loop(0, nch)
            def _(i):
                b = base_e + i * CS
                pltpu.sync_copy(idx_hbm.at[pl.ds(b, CS)], idx_v)
                pltpu.sync_copy(vals_hbm.at[plane, pl.ds(b, CS)], val_v)
                pltpu.sync_copy(val_v, acc_sh.at[idx_v], add=True)

            plsc.subcore_barrier()

            for kk in range(2):
                k = s + NS * kk

                @pl.when(k < NZCH)
                def _():
                    pltpu.sync_copy(
                        acc_sh.at[pl.ds(k * ZR, ZR)],
                        out_hbm.at[plane, pl.ds(k * ZR, ZR)],
                    )

    return pl.kernel(
        body,
        mesh=_sc_mesh(),
        out_type=jax.ShapeDtypeStruct((4, N, H), jnp.float32),
        scratch_types=[
            pltpu.VMEM((CS,), jnp.int32),
            pltpu.VMEM((CS, H), jnp.float32),
            pltpu.VMEM_SHARED((N, H), jnp.float32),
        ],
    )


def _pack_pairs(a_bf):
    k = a_bf.shape[1] // 2
    lo = lax.bitcast_convert_type(a_bf[:, :k], jnp.uint16).astype(jnp.uint32)
    hi = lax.bitcast_convert_type(a_bf[:, k:], jnp.uint16).astype(jnp.uint32)
    return lax.bitcast_convert_type((hi << 16) | lo, jnp.float32)


def _unpack_pairs(p):
    xi = lax.bitcast_convert_type(p, jnp.int32)
    lo = lax.bitcast_convert_type(xi << 16, jnp.float32)
    hi = lax.bitcast_convert_type(xi & jnp.int32(-65536), jnp.float32)
    return jnp.concatenate([lo, hi], axis=1)


def _tc_dense_body(wT_ref, rbfT_ref, xvj_ref, evT_ref,
                   xw1_ref, xb1_ref, xw2_ref, xb2_ref,
                   rw_ref, rb_ref, iw1_ref, ib1_ref, iw2_ref, ib2_ref,
                   out_ref):
    f32 = jnp.float32
    bf = jnp.bfloat16

    def mm(a, b):
        return jnp.dot(a.astype(bf), b.astype(bf), preferred_element_type=f32)

    def mm_t(a, b):
        return lax.dot_general(a.astype(bf), b.astype(bf),
                               (((0,), (0,)), ((), ())),
                               preferred_element_type=f32)

    xj = _unpack_pairs(xvj_ref[:, :H // 2])
    vj = _unpack_pairs(xvj_ref[:, H // 2:])
    h1 = mm(xj, xw1_ref[...]) + xb1_ref[...]
    a1 = h1 * lax.logistic(h1)
    xh = mm(a1, xw2_ref[...]) + xb2_ref[...]
    rbfh = mm_t(rbfT_ref[...], rw_ref[...]) + rb_ref[...]
    g1 = mm_t(wT_ref[...], iw1_ref[...]) + ib1_ref[...]
    ga = g1 * lax.logistic(g1)
    g2 = mm(ga, iw2_ref[...]) + ib2_ref[...]
    m = xh * (rbfh * g2)
    x1 = m[:, :H]
    xh2 = m[:, H:2 * H] * INV_SQRT_3
    xh3 = m[:, 2 * H:]
    out_ref[0, :, :] = x1
    for c in range(3):
        ev_c = jnp.transpose(evT_ref[c:c + 1, :])
        vj_c = vj[:, c * H:(c + 1) * H]
        out_ref[1 + c, :, :] = (vj_c * xh2 + ev_c * xh3) * INV_SQRT_H


def _tc_dense(base, seg, weightT, edge_rbfT, xv_j, evT,
              xp_w1, xp_b1, xp_w2, xp_b2, rbf_w, rbf_b,
              inv_w1, inv_b1, inv_w2, inv_b2):
    grid = (seg // EB,)
    bb = base // EB
    edge_spec = lambda d: pl.BlockSpec((EB, d), lambda i: (i, 0))
    edge_spec_t = lambda d: pl.BlockSpec((d, EB), lambda i: (0, i + bb))
    full_spec = lambda a, b: pl.BlockSpec((a, b), lambda i: (0, 0))
    return pl.pallas_call(
        _tc_dense_body,
        grid=grid,
        in_specs=[
            edge_spec_t(3 * H + R),
            edge_spec_t(R),
            edge_spec(2 * H),
            edge_spec_t(3),
            full_spec(H, H),
            full_spec(1, H),
            full_spec(H, 3 * H),
            full_spec(1, 3 * H),
            full_spec(R, 3 * H),
            full_spec(1, 3 * H),
            full_spec(3 * H + R, 3 * H),
            full_spec(1, 3 * H),
            full_spec(3 * H, 3 * H),
            full_spec(1, 3 * H),
        ],
        out_specs=pl.BlockSpec((4, EB, H), lambda i: (0, i, 0)),
        out_shape=jax.ShapeDtypeStruct((4, seg, H), jnp.float32),
        compiler_params=pltpu.CompilerParams(
            dimension_semantics=("parallel",)),
    )(weightT, edge_rbfT, xv_j, evT,
      xp_w1, xp_b1.reshape(1, H), xp_w2, xp_b2.reshape(1, 3 * H),
      rbf_w, rbf_b.reshape(1, 3 * H),
      inv_w1, inv_b1.reshape(1, 3 * H), inv_w2, inv_b2.reshape(1, 3 * H))


@jax.jit
def _impl(x, vec, edge_index, edge_rbf, weight, edge_vector,
          xp_w1, xp_b1, xp_w2, xp_b2, rbf_w, rbf_b,
          inv_w1, inv_b1, inv_w2, inv_b2):
    src = edge_index[0].astype(jnp.int32)
    dst = edge_index[1].astype(jnp.int32)
    vec_flat = vec.reshape(N, 3 * H)
    x_pk = _pack_pairs(x.astype(jnp.bfloat16))
    vec_pk = _pack_pairs(vec_flat.astype(jnp.bfloat16))
    tab = jnp.concatenate([x_pk, vec_pk], axis=1)
    acc = jnp.zeros((4, N, H), jnp.float32)
    base = 0
    for seg in SEGS:
        sl = slice(base, base + seg)
        xv_k = _sc_gather_kernel(seg)(tab, src[sl])
        planes_k = _tc_dense(base, seg, weight.T, edge_rbf.T, xv_k,
                             edge_vector.T,
                             xp_w1, xp_b1, xp_w2, xp_b2, rbf_w, rbf_b,
                             inv_w1, inv_b1, inv_w2, inv_b2)
        acc = _sc_scatter_kernel(seg)(planes_k, dst[sl], acc)
        base += seg
    dx = acc[0]
    dvec = jnp.stack((acc[1], acc[2], acc[3]), axis=1)
    return dx, dvec


def kernel(x, vec, edge_index, edge_rbf, weight, edge_vector,
           xp_w1, xp_b1, xp_w2, xp_b2, rbf_w, rbf_b,
           inv_w1, inv_b1, inv_w2, inv_b2):
    return _impl(x, vec, edge_index, edge_rbf, weight, edge_vector,
                 xp_w1, xp_b1, xp_w2, xp_b2, rbf_w, rbf_b,
                 inv_w1, inv_b1, inv_w2, inv_b2)


# speedup vs baseline: 28.1389x; 1.0482x over previous
import functools
import math

import jax
import jax.numpy as jnp
from jax import lax
from jax.experimental import pallas as pl
from jax.experimental.pallas import tpu as pltpu
from jax.experimental.pallas import tpu_sc as plsc

N = 10000
E = 320000
H = 128
R = 32
INV_SQRT_3 = 1.0 / math.sqrt(3.0)
INV_SQRT_H = 1.0 / math.sqrt(H)

NW = 32
CG = 200

NS = 16
CS = 160
ZR = 400
NZCH = N // ZR

EB = 640
SEGS = (12800, 64000, 76800, 76800, 76800, 12800)


@functools.cache
def _sc_mesh():
    return plsc.VectorSubcoreMesh(
        core_axis_name="c", subcore_axis_name="s", num_cores=2, num_subcores=16
    )


@functools.cache
def _sc_gather_kernel(seg):
    per_w = seg // NW
    nch = per_w // CG

    def body(tab_hbm, idx_hbm, out_hbm, idx_v0, idx_v1, row_v0, row_v1,
             sem_i0, sem_i1, sem_w0, sem_w1):
        wid = lax.axis_index("s") * 2 + lax.axis_index("c")
        base = wid * per_w
        idx_vs = (idx_v0, idx_v1)
        row_vs = (row_v0, row_v1)
        sem_is = (sem_i0, sem_i1)
        sem_ws = (sem_w0, sem_w1)

        def fire_idx(i):
            b = base + i * CG
            return pltpu.async_copy(idx_hbm.at[pl.ds(b, CG)],
                                    idx_vs[i % 2], sem_is[i % 2])

        pend = [fire_idx(0), fire_idx(1) if nch > 1 else None]
        wb = [None, None]
        for i in range(nch):
            bf = i % 2
            pend[bf].wait()
            if wb[bf] is not None:
                wb[bf].wait()
            pltpu.sync_copy(tab_hbm.at[idx_vs[bf]], row_vs[bf])
            b = base + i * CG
            wb[bf] = pltpu.async_copy(row_vs[bf], out_hbm.at[pl.ds(b, CG)],
                                      sem_ws[bf])
            if i + 2 < nch:
                pend[bf] = fire_idx(i + 2)
        for bf in range(2):
            if wb[bf] is not None:
                wb[bf].wait()

    return pl.kernel(
        body,
        mesh=_sc_mesh(),
        out_type=jax.ShapeDtypeStruct((seg, 2 * H), jnp.float32),
        scratch_types=[
            pltpu.VMEM((CG,), jnp.int32),
            pltpu.VMEM((CG,), jnp.int32),
            pltpu.VMEM((CG, 2 * H), jnp.float32),
            pltpu.VMEM((CG, 2 * H), jnp.float32),
            pltpu.SemaphoreType.DMA,
            pltpu.SemaphoreType.DMA,
            pltpu.SemaphoreType.DMA,
            pltpu.SemaphoreType.DMA,
        ],
    )


@functools.cache
def _sc_scatter_kernel(seg):
    per_s = seg // NS
    nch = per_s // CS

    def body(vals_hbm, idx_hbm, init_hbm, out_hbm,
             idx_v0, idx_v1, val_v0, val_v1, sem0, sem1, acc_sh):
        core = lax.axis_index("c")
        s = lax.axis_index("s")
        idx_vs = (idx_v0, idx_v1)
        val_vs = (val_v0, val_v1)
        sems = (sem0, sem1)
        for p in range(2):
            plane = core * 2 + p
            base_e = s * per_s

            def fire(i):
                b = base_e + i * CS
                bf = i % 2
                c1 = pltpu.async_copy(idx_hbm.at[pl.ds(b, CS)],
                                      idx_vs[bf], sems[bf])
                c2 = pltpu.async_copy(vals_hbm.at[plane, pl.ds(b, CS)],
                                      val_vs[bf], sems[bf])
                return (c1, c2)

            pend = [fire(0), fire(1) if nch > 1 else None]

            for kk in range(2):
                k = s + NS * kk

                @pl.when(k < NZCH)
                def _():
                    pltpu.sync_copy(init_hbm.at[plane, pl.ds(k * ZR, ZR)],
                                    acc_sh.at[pl.ds(k * ZR, ZR)])

            plsc.subcore_barrier()

            for i in range(nch):
                bf = i % 2
                c1, c2 = pend[bf]
                c1.wait()
                c2.wait()
                pltpu.sync_copy(val_vs[bf], acc_sh.at[idx_vs[bf]], add=True)
                if i + 2 < nch:
                    pend[bf] = fire(i + 2)

            plsc.subcore_barrier()

            for kk in range(2):
                k = s + NS * kk

                @pl.when(k < NZCH)
                def _():
                    pltpu.sync_copy(
                        acc_sh.at[pl.ds(k * ZR, ZR)],
                        out_hbm.at[plane, pl.ds(k * ZR, ZR)],
                    )

    return pl.kernel(
        body,
        mesh=_sc_mesh(),
        out_type=jax.ShapeDtypeStruct((4, N, H), jnp.float32),
        scratch_types=[
            pltpu.VMEM((CS,), jnp.int32),
            pltpu.VMEM((CS,), jnp.int32),
            pltpu.VMEM((CS, H), jnp.float32),
            pltpu.VMEM((CS, H), jnp.float32),
            pltpu.SemaphoreType.DMA,
            pltpu.SemaphoreType.DMA,
            pltpu.VMEM_SHARED((N, H), jnp.float32),
        ],
    )


def _pack_pairs(a_bf):
    k = a_bf.shape[1] // 2
    lo = lax.bitcast_convert_type(a_bf[:, :k], jnp.uint16).astype(jnp.uint32)
    hi = lax.bitcast_convert_type(a_bf[:, k:], jnp.uint16).astype(jnp.uint32)
    return lax.bitcast_convert_type((hi << 16) | lo, jnp.float32)


def _unpack_pairs(p):
    xi = lax.bitcast_convert_type(p, jnp.int32)
    lo = lax.bitcast_convert_type(xi << 16, jnp.float32)
    hi = lax.bitcast_convert_type(xi & jnp.int32(-65536), jnp.float32)
    return jnp.concatenate([lo, hi], axis=1)


def _tc_dense_body(wT_ref, rbfT_ref, xvj_ref, evT_ref,
                   xw1_ref, xb1_ref, xw2_ref, xb2_ref,
                   rw_ref, rb_ref, iw1_ref, ib1_ref, iw2_ref, ib2_ref,
                   out_ref):
    f32 = jnp.float32
    bf = jnp.bfloat16

    def mm(a, b):
        return jnp.dot(a.astype(bf), b.astype(bf), preferred_element_type=f32)

    def mm_t(a, b):
        return lax.dot_general(a.astype(bf), b.astype(bf),
                               (((0,), (0,)), ((), ())),
                               preferred_element_type=f32)

    xj = _unpack_pairs(xvj_ref[:, :H // 2])
    vj = _unpack_pairs(xvj_ref[:, H // 2:])
    h1 = mm(xj, xw1_ref[...]) + xb1_ref[...]
    a1 = h1 * lax.logistic(h1)
    xh = mm(a1, xw2_ref[...]) + xb2_ref[...]
    rbfh = mm_t(rbfT_ref[...], rw_ref[...]) + rb_ref[...]
    g1 = mm_t(wT_ref[...], iw1_ref[...]) + ib1_ref[...]
    ga = g1 * lax.logistic(g1)
    g2 = mm(ga, iw2_ref[...]) + ib2_ref[...]
    m = xh * (rbfh * g2)
    x1 = m[:, :H]
    xh2 = m[:, H:2 * H] * INV_SQRT_3
    xh3 = m[:, 2 * H:]
    out_ref[0, :, :] = x1
    for c in range(3):
        ev_c = jnp.transpose(evT_ref[c:c + 1, :])
        vj_c = vj[:, c * H:(c + 1) * H]
        out_ref[1 + c, :, :] = (vj_c * xh2 + ev_c * xh3) * INV_SQRT_H


def _tc_dense(base, seg, weightT, edge_rbfT, xv_j, evT,
              xp_w1, xp_b1, xp_w2, xp_b2, rbf_w, rbf_b,
              inv_w1, inv_b1, inv_w2, inv_b2):
    grid = (seg // EB,)
    bb = base // EB
    edge_spec = lambda d: pl.BlockSpec((EB, d), lambda i: (i, 0))
    edge_spec_t = lambda d: pl.BlockSpec((d, EB), lambda i: (0, i + bb))
    full_spec = lambda a, b: pl.BlockSpec((a, b), lambda i: (0, 0))
    return pl.pallas_call(
        _tc_dense_body,
        grid=grid,
        in_specs=[
            edge_spec_t(3 * H + R),
            edge_spec_t(R),
            edge_spec(2 * H),
            edge_spec_t(3),
            full_spec(H, H),
            full_spec(1, H),
            full_spec(H, 3 * H),
            full_spec(1, 3 * H),
            full_spec(R, 3 * H),
            full_spec(1, 3 * H),
            full_spec(3 * H + R, 3 * H),
            full_spec(1, 3 * H),
            full_spec(3 * H, 3 * H),
            full_spec(1, 3 * H),
        ],
        out_specs=pl.BlockSpec((4, EB, H), lambda i: (0, i, 0)),
        out_shape=jax.ShapeDtypeStruct((4, seg, H), jnp.float32),
        compiler_params=pltpu.CompilerParams(
            dimension_semantics=("parallel",)),
    )(weightT, edge_rbfT, xv_j, evT,
      xp_w1, xp_b1.reshape(1, H), xp_w2, xp_b2.reshape(1, 3 * H),
      rbf_w, rbf_b.reshape(1, 3 * H),
      inv_w1, inv_b1.reshape(1, 3 * H), inv_w2, inv_b2.reshape(1, 3 * H))


@jax.jit
def _impl(x, vec, edge_index, edge_rbf, weight, edge_vector,
          xp_w1, xp_b1, xp_w2, xp_b2, rbf_w, rbf_b,
          inv_w1, inv_b1, inv_w2, inv_b2):
    src = edge_index[0].astype(jnp.int32)
    dst = edge_index[1].astype(jnp.int32)
    vec_flat = vec.reshape(N, 3 * H)
    x_pk = _pack_pairs(x.astype(jnp.bfloat16))
    vec_pk = _pack_pairs(vec_flat.astype(jnp.bfloat16))
    tab = jnp.concatenate([x_pk, vec_pk], axis=1)
    acc = jnp.zeros((4, N, H), jnp.float32)
    base = 0
    for seg in SEGS:
        sl = slice(base, base + seg)
        xv_k = _sc_gather_kernel(seg)(tab, src[sl])
        planes_k = _tc_dense(base, seg, weight.T, edge_rbf.T, xv_k,
                             edge_vector.T,
                             xp_w1, xp_b1, xp_w2, xp_b2, rbf_w, rbf_b,
                             inv_w1, inv_b1, inv_w2, inv_b2)
        acc = _sc_scatter_kernel(seg)(planes_k, dst[sl], acc)
        base += seg
    dx = acc[0]
    dvec = jnp.stack((acc[1], acc[2], acc[3]), axis=1)
    return dx, dvec


def kernel(x, vec, edge_index, edge_rbf, weight, edge_vector,
           xp_w1, xp_b1, xp_w2, xp_b2, rbf_w, rbf_b,
           inv_w1, inv_b1, inv_w2, inv_b2):
    return _impl(x, vec, edge_index, edge_rbf, weight, edge_vector,
                 xp_w1, xp_b1, xp_w2, xp_b2, rbf_w, rbf_b,
                 inv_w1, inv_b1, inv_w2, inv_b2)


# speedup vs baseline: 28.8008x; 1.0235x over previous
import functools
import math

import jax
import jax.numpy as jnp
from jax import lax
from jax.experimental import pallas as pl
from jax.experimental.pallas import tpu as pltpu
from jax.experimental.pallas import tpu_sc as plsc

N = 10000
E = 320000
H = 128
R = 32
INV_SQRT_3 = 1.0 / math.sqrt(3.0)
INV_SQRT_H = 1.0 / math.sqrt(H)

NW = 32
CG = 200

NS = 16
CS = 160
ZR = 400
NZCH = N // ZR

EB = 640
SEGS = (12800, 64000, 76800, 76800, 64000, 25600)


@functools.cache
def _sc_mesh():
    return plsc.VectorSubcoreMesh(
        core_axis_name="c", subcore_axis_name="s", num_cores=2, num_subcores=16
    )


@functools.cache
def _sc_gather_kernel(seg):
    per_w = seg // NW
    nch = per_w // CG

    def body(tab_hbm, idx_hbm, out_hbm, idx_v0, idx_v1, row_v0, row_v1,
             sem_i0, sem_i1, sem_w0, sem_w1):
        wid = lax.axis_index("s") * 2 + lax.axis_index("c")
        base = wid * per_w
        idx_vs = (idx_v0, idx_v1)
        row_vs = (row_v0, row_v1)
        sem_is = (sem_i0, sem_i1)
        sem_ws = (sem_w0, sem_w1)

        def fire_idx(i):
            b = base + i * CG
            return pltpu.async_copy(idx_hbm.at[pl.ds(b, CG)],
                                    idx_vs[i % 2], sem_is[i % 2])

        pend = [fire_idx(0), fire_idx(1) if nch > 1 else None]
        wb = [None, None]
        for i in range(nch):
            bf = i % 2
            pend[bf].wait()
            if wb[bf] is not None:
                wb[bf].wait()
            pltpu.sync_copy(tab_hbm.at[idx_vs[bf]], row_vs[bf])
            b = base + i * CG
            wb[bf] = pltpu.async_copy(row_vs[bf], out_hbm.at[pl.ds(b, CG)],
                                      sem_ws[bf])
            if i + 2 < nch:
                pend[bf] = fire_idx(i + 2)
        for bf in range(2):
            if wb[bf] is not None:
                wb[bf].wait()

    return pl.kernel(
        body,
        mesh=_sc_mesh(),
        out_type=jax.ShapeDtypeStruct((seg, 2 * H), jnp.float32),
        scratch_types=[
            pltpu.VMEM((CG,), jnp.int32),
            pltpu.VMEM((CG,), jnp.int32),
            pltpu.VMEM((CG, 2 * H), jnp.float32),
            pltpu.VMEM((CG, 2 * H), jnp.float32),
            pltpu.SemaphoreType.DMA,
            pltpu.SemaphoreType.DMA,
            pltpu.SemaphoreType.DMA,
            pltpu.SemaphoreType.DMA,
        ],
    )


@functools.cache
def _sc_scatter_kernel(seg):
    per_s = seg // NS
    nch = per_s // CS

    def body(vals_hbm, idx_hbm, init_hbm, out_hbm,
             idx_v0, idx_v1, val_v0, val_v1, sem0, sem1, acc_sh):
        core = lax.axis_index("c")
        s = lax.axis_index("s")
        idx_vs = (idx_v0, idx_v1)
        val_vs = (val_v0, val_v1)
        sems = (sem0, sem1)
        for p in range(2):
            plane = core * 2 + p
            base_e = s * per_s

            def fire(i):
                b = base_e + i * CS
                bf = i % 2
                c1 = pltpu.async_copy(idx_hbm.at[pl.ds(b, CS)],
                                      idx_vs[bf], sems[bf])
                c2 = pltpu.async_copy(vals_hbm.at[plane, pl.ds(b, CS)],
                                      val_vs[bf], sems[bf])
                return (c1, c2)

            pend = [fire(0), fire(1) if nch > 1 else None]

            for kk in range(2):
                k = s + NS * kk

                @pl.when(k < NZCH)
                def _():
                    pltpu.sync_copy(init_hbm.at[plane, pl.ds(k * ZR, ZR)],
                                    acc_sh.at[pl.ds(k * ZR, ZR)])

            plsc.subcore_barrier()

            for i in range(nch):
                bf = i % 2
                c1, c2 = pend[bf]
                c1.wait()
                c2.wait()
                pltpu.sync_copy(val_vs[bf], acc_sh.at[idx_vs[bf]], add=True)
                if i + 2 < nch:
                    pend[bf] = fire(i + 2)

            plsc.subcore_barrier()

            for kk in range(2):
                k = s + NS * kk

                @pl.when(k < NZCH)
                def _():
                    pltpu.sync_copy(
                        acc_sh.at[pl.ds(k * ZR, ZR)],
                        out_hbm.at[plane, pl.ds(k * ZR, ZR)],
                    )

    return pl.kernel(
        body,
        mesh=_sc_mesh(),
        out_type=jax.ShapeDtypeStruct((4, N, H), jnp.float32),
        scratch_types=[
            pltpu.VMEM((CS,), jnp.int32),
            pltpu.VMEM((CS,), jnp.int32),
            pltpu.VMEM((CS, H), jnp.float32),
            pltpu.VMEM((CS, H), jnp.float32),
            pltpu.SemaphoreType.DMA,
            pltpu.SemaphoreType.DMA,
            pltpu.VMEM_SHARED((N, H), jnp.float32),
        ],
    )


def _pack_pairs(a_bf):
    k = a_bf.shape[1] // 2
    lo = lax.bitcast_convert_type(a_bf[:, :k], jnp.uint16).astype(jnp.uint32)
    hi = lax.bitcast_convert_type(a_bf[:, k:], jnp.uint16).astype(jnp.uint32)
    return lax.bitcast_convert_type((hi << 16) | lo, jnp.float32)


def _unpack_pairs(p):
    xi = lax.bitcast_convert_type(p, jnp.int32)
    lo = lax.bitcast_convert_type(xi << 16, jnp.float32)
    hi = lax.bitcast_convert_type(xi & jnp.int32(-65536), jnp.float32)
    return jnp.concatenate([lo, hi], axis=1)


def _tc_dense_body(wT_ref, rbfT_ref, xvj_ref, evT_ref,
                   xw1_ref, xb1_ref, xw2_ref, xb2_ref,
                   rw_ref, rb_ref, iw1_ref, ib1_ref, iw2_ref, ib2_ref,
                   out_ref):
    f32 = jnp.float32
    bf = jnp.bfloat16

    def mm(a, b):
        return jnp.dot(a.astype(bf), b.astype(bf), preferred_element_type=f32)

    def mm_t(a, b):
        return lax.dot_general(a.astype(bf), b.astype(bf),
                               (((0,), (0,)), ((), ())),
                               preferred_element_type=f32)

    xj = _unpack_pairs(xvj_ref[:, :H // 2])
    vj = _unpack_pairs(xvj_ref[:, H // 2:])
    h1 = mm(xj, xw1_ref[...]) + xb1_ref[...]
    a1 = h1 * lax.logistic(h1)
    xh = mm(a1, xw2_ref[...]) + xb2_ref[...]
    rbfh = mm_t(rbfT_ref[...], rw_ref[...]) + rb_ref[...]
    g1 = mm_t(wT_ref[...], iw1_ref[...]) + ib1_ref[...]
    ga = g1 * lax.logistic(g1)
    g2 = mm(ga, iw2_ref[...]) + ib2_ref[...]
    m = xh * (rbfh * g2)
    x1 = m[:, :H]
    xh2 = m[:, H:2 * H] * INV_SQRT_3
    xh3 = m[:, 2 * H:]
    out_ref[0, :, :] = x1
    for c in range(3):
        ev_c = jnp.transpose(evT_ref[c:c + 1, :])
        vj_c = vj[:, c * H:(c + 1) * H]
        out_ref[1 + c, :, :] = (vj_c * xh2 + ev_c * xh3) * INV_SQRT_H


def _tc_dense(base, seg, weightT, edge_rbfT, xv_j, evT,
              xp_w1, xp_b1, xp_w2, xp_b2, rbf_w, rbf_b,
              inv_w1, inv_b1, inv_w2, inv_b2):
    grid = (seg // EB,)
    bb = base // EB
    edge_spec = lambda d: pl.BlockSpec((EB, d), lambda i: (i, 0))
    edge_spec_t = lambda d: pl.BlockSpec((d, EB), lambda i: (0, i + bb))
    full_spec = lambda a, b: pl.BlockSpec((a, b), lambda i: (0, 0))
    return pl.pallas_call(
        _tc_dense_body,
        grid=grid,
        in_specs=[
            edge_spec_t(3 * H + R),
            edge_spec_t(R),
            edge_spec(2 * H),
            edge_spec_t(3),
            full_spec(H, H),
            full_spec(1, H),
            full_spec(H, 3 * H),
            full_spec(1, 3 * H),
            full_spec(R, 3 * H),
            full_spec(1, 3 * H),
            full_spec(3 * H + R, 3 * H),
            full_spec(1, 3 * H),
            full_spec(3 * H, 3 * H),
            full_spec(1, 3 * H),
        ],
        out_specs=pl.BlockSpec((4, EB, H), lambda i: (0, i, 0)),
        out_shape=jax.ShapeDtypeStruct((4, seg, H), jnp.float32),
        compiler_params=pltpu.CompilerParams(
            dimension_semantics=("parallel",)),
    )(weightT, edge_rbfT, xv_j, evT,
      xp_w1, xp_b1.reshape(1, H), xp_w2, xp_b2.reshape(1, 3 * H),
      rbf_w, rbf_b.reshape(1, 3 * H),
      inv_w1, inv_b1.reshape(1, 3 * H), inv_w2, inv_b2.reshape(1, 3 * H))


@jax.jit
def _impl(x, vec, edge_index, edge_rbf, weight, edge_vector,
          xp_w1, xp_b1, xp_w2, xp_b2, rbf_w, rbf_b,
          inv_w1, inv_b1, inv_w2, inv_b2):
    src = edge_index[0].astype(jnp.int32)
    dst = edge_index[1].astype(jnp.int32)
    vec_flat = vec.reshape(N, 3 * H)
    x_pk = _pack_pairs(x.astype(jnp.bfloat16))
    vec_pk = _pack_pairs(vec_flat.astype(jnp.bfloat16))
    tab = jnp.concatenate([x_pk, vec_pk], axis=1)
    acc = jnp.zeros((4, N, H), jnp.float32)
    base = 0
    for seg in SEGS:
        sl = slice(base, base + seg)
        xv_k = _sc_gather_kernel(seg)(tab, src[sl])
        planes_k = _tc_dense(base, seg, weight.T, edge_rbf.T, xv_k,
                             edge_vector.T,
                             xp_w1, xp_b1, xp_w2, xp_b2, rbf_w, rbf_b,
                             inv_w1, inv_b1, inv_w2, inv_b2)
        acc = _sc_scatter_kernel(seg)(planes_k, dst[sl], acc)
        base += seg
    dx = acc[0]
    dvec = jnp.stack((acc[1], acc[2], acc[3]), axis=1)
    return dx, dvec


def kernel(x, vec, edge_index, edge_rbf, weight, edge_vector,
           xp_w1, xp_b1, xp_w2, xp_b2, rbf_w, rbf_b,
           inv_w1, inv_b1, inv_w2, inv_b2):
    return _impl(x, vec, edge_index, edge_rbf, weight, edge_vector,
                 xp_w1, xp_b1, xp_w2, xp_b2, rbf_w, rbf_b,
                 inv_w1, inv_b1, inv_w2, inv_b2)
